# baseline (speedup 1.0000x reference)
import functools

import jax
import jax.numpy as jnp
from jax import lax
from jax.experimental import pallas as pl
from jax.experimental.pallas import tpu as pltpu
from jax.experimental.pallas import tpu_sc as plsc

_K = 32
_L = 16
_NW = 32


def _sc_body(N, T, params_hbm, ch_hbm, rw_hbm, out_hbm,
             params_v, ch_v, rw_v, stage_v, sem):
    wid = lax.axis_index("s") * 2 + lax.axis_index("c")
    rows_per_w = N // _NW
    groups = rows_per_w // _L
    row_len = T * _K
    slab = _L * row_len

    pltpu.sync_copy(params_hbm, params_v)
    iv = params_v[pl.ds(0, _L)]
    ap = params_v[pl.ds(_L, _L)]
    am = params_v[pl.ds(2 * _L, _L)]
    iota = lax.iota(jnp.int32, _L)
    iota_rows = iota * row_len

    for g in range(groups):
        rows = wid * rows_per_w + g * _L
        pltpu.sync_copy(ch_hbm.at[pl.ds(rows, _L), :], ch_v)
        pltpu.sync_copy(rw_hbm.at[pl.ds(rows, _L), :], rw_v)

        for l in range(_L):
            for j in range(2):
                stage_v[pl.ds(l * row_len + j * _L, _L)] = iv

        def step(t, carry):
            t_vec = jnp.full((_L,), t, jnp.int32)
            ch = plsc.load_gather(ch_v, [iota, t_vec])
            rw = plsc.load_gather(rw_v, [iota, t_vec])
            col = t * _K
            kpos = iota_rows + col + ch
            chosen = plsc.load_gather(stage_v, [kpos])
            pe = rw - chosen
            pe = jnp.where(rw != rw, 0.0, pe)
            coef = jnp.where(pe >= 0, ap, am)
            upd = chosen + coef * pe
            for l in range(_L):
                for j in range(2):
                    stage_v[pl.ds(l * row_len + col + _K + j * _L, _L)] = (
                        stage_v[pl.ds(l * row_len + col + j * _L, _L)])
            plsc.store_scatter(stage_v, [kpos + _K], upd)
            return carry

        lax.fori_loop(0, T - 1, step, 0)

        pltpu.async_copy(stage_v, out_hbm.at[pl.ds(rows * row_len, slab)],
                         sem).wait()


def kernel(choices, rewards, alpha_plus, alpha_minus, initial_values):
    N, T = choices.shape
    iv = 100.0 * jnp.tanh(initial_values)
    ap = jax.nn.sigmoid(alpha_plus)
    am = jax.nn.sigmoid(alpha_minus)
    params = jnp.concatenate([
        jnp.full((_L,), iv, jnp.float32),
        jnp.full((_L,), ap, jnp.float32),
        jnp.full((_L,), am, jnp.float32),
    ])

    mesh = plsc.VectorSubcoreMesh(core_axis_name="c", subcore_axis_name="s")
    run = pl.kernel(
        functools.partial(_sc_body, N, T),
        out_type=jax.ShapeDtypeStruct((N * T * _K,), jnp.float32),
        mesh=mesh,
        scratch_types=[
            pltpu.VMEM((3 * _L,), jnp.float32),
            pltpu.VMEM((_L, T), jnp.int32),
            pltpu.VMEM((_L, T), jnp.float32),
            pltpu.VMEM((_L * T * _K,), jnp.float32),
            pltpu.SemaphoreType.DMA,
        ],
        compiler_params=pltpu.CompilerParams(
            use_tc_tiling_on_sc=False, needs_layout_passes=False),
    )
    return run(params, choices, rewards).reshape(N, T, _K)
